# baseline (speedup 1.0000x reference)
import functools

import jax
import jax.numpy as jnp
from jax import lax
from jax.experimental import pallas as pl
from jax.experimental.pallas import tpu as pltpu
from jax.experimental.pallas import tpu_sc as plsc

_E = 768
_P = 166
_PP = 168
_B = 16
_S = 1126
_W = 5
_CH = 64
_NCH = 18
_SCROWS = 1120
_TAILW = _SCROWS - (_NCH - 1) * _CH
_FIX = _S - _SCROWS
_NW = 32
_PERW = (_B * _NCH) // _NW


def _comb_kernel(csid_ref, data_ref, w_ref, out_ref):
    del csid_ref
    x = data_ref[0]
    snip = x[:, :_E]
    wt = w_ref[...]
    shifted = (
        jnp.concatenate([snip[:1], snip[:1], snip[:-2]], axis=0),
        jnp.concatenate([snip[:1], snip[:-1]], axis=0),
        snip,
        jnp.concatenate([snip[1:], snip[-1:]], axis=0),
        jnp.concatenate([snip[2:], snip[-1:], snip[-1:]], axis=0),
    )
    acc = shifted[0] * wt[0:1, :]
    for k in range(1, _W):
        acc = acc + shifted[k] * wt[k:k + 1, :]
    out_ref[...] = jnp.concatenate([acc, acc[-2:]], axis=0)


def _build_comb(csid, data_r, w):
    grid_spec = pltpu.PrefetchScalarGridSpec(
        num_scalar_prefetch=1,
        grid=(_B,),
        in_specs=[
            pl.BlockSpec((1, _P, 2 * _E), lambda b, csid_ref: (csid_ref[b], 0, 0)),
            pl.BlockSpec((_W, _E), lambda b, csid_ref: (0, 0)),
        ],
        out_specs=pl.BlockSpec((_PP, _E), lambda b, csid_ref: (b, 0)),
    )
    return pl.pallas_call(
        _comb_kernel,
        grid_spec=grid_spec,
        out_shape=jax.ShapeDtypeStruct((_B * _PP, _E), jnp.float32),
    )(csid, data_r, w)


def _sc_gather_body(comb_hbm, inpc_hbm, out_hbm, idx_v, gidx_v,
                    rows0, rows1, gsem, wsem0, wsem1):
    cid = lax.axis_index("c")
    sid = lax.axis_index("s")
    wid = cid * 16 + sid
    b = wid // 2
    c0 = (wid % 2) * _PERW
    bufs = (rows0, rows1)
    wsems = (wsem0, wsem1)
    write_cps = []
    for i in range(_PERW):
        c = c0 + i
        s0 = c * _CH
        buf = bufs[i % 2]
        if i >= 2:
            write_cps[i - 2].wait()
        pltpu.sync_copy(inpc_hbm.at[b * _NCH + c], idx_v)
        for j in range(_CH // 16):
            v = idx_v[0, pl.ds(j * 16, 16)]
            v = jnp.clip(v, 0, _P - 1) + b * _PP
            gidx_v[pl.ds(j * 16, 16)] = v
        pltpu.async_copy(comb_hbm.at[gidx_v], buf, gsem).wait()
        if i < _PERW - 1:
            cp = pltpu.make_async_copy(
                buf, out_hbm.at[b, pl.ds(s0, _CH)], wsems[i % 2])
            cp.start()
            write_cps.append(cp)
        else:
            @pl.when(c == _NCH - 1)
            def _():
                pltpu.sync_copy(buf.at[pl.ds(0, _TAILW)],
                                out_hbm.at[b, pl.ds(s0, _TAILW)])

            @pl.when(c != _NCH - 1)
            def _():
                pltpu.sync_copy(buf, out_hbm.at[b, pl.ds(s0, _CH)])
    write_cps[-1].wait()


_sc_gather_cache = []


def _sc_gather():
    if not _sc_gather_cache:
        _sc_gather_cache.append(functools.partial(
            pl.kernel,
            out_type=jax.ShapeDtypeStruct((_B, _S, _E), jnp.float32),
            mesh=plsc.VectorSubcoreMesh(core_axis_name="c", subcore_axis_name="s"),
            scratch_types=[
                pltpu.VMEM((1, _CH), jnp.int32),
                pltpu.VMEM((_CH,), jnp.int32),
                pltpu.VMEM((_CH, _E), jnp.float32),
                pltpu.VMEM((_CH, _E), jnp.float32),
                pltpu.SemaphoreType.DMA,
                pltpu.SemaphoreType.DMA,
                pltpu.SemaphoreType.DMA,
            ],
        )(_sc_gather_body))
    return _sc_gather_cache[0]


def _fixup_kernel(idx_ref, comb_ref, outp_ref, out_ref):
    del outp_ref
    idx = jnp.clip(idx_ref[0, 0, :], 0, _P - 1)
    pos = lax.broadcasted_iota(jnp.int32, (8, _PP), 1)
    oh = (pos == idx[:, None]).astype(jnp.float32)
    out_ref[...] = jnp.dot(oh, comb_ref[...],
                           preferred_element_type=jnp.float32)[None]


def _fixup(idx6, comb2, outp):
    return pl.pallas_call(
        _fixup_kernel,
        grid=(_B,),
        in_specs=[
            pl.BlockSpec((1, 1, 8), lambda b: (b, 0, 0)),
            pl.BlockSpec((_PP, _E), lambda b: (b, 0)),
            pl.BlockSpec((1, 8, _E), lambda b: (b, _SCROWS // 8, 0)),
        ],
        out_specs=pl.BlockSpec((1, 8, _E), lambda b: (b, _SCROWS // 8, 0)),
        out_shape=jax.ShapeDtypeStruct((_B, _S, _E), jnp.float32),
        input_output_aliases={2: 0},
    )(idx6, comb2, outp)


def kernel(inputs, code_snippet_id, data, w):
    inputs = inputs.astype(jnp.int32)
    csid = code_snippet_id.astype(jnp.int32).reshape(_B)
    data_r = data.reshape(data.shape[0], _P, 2 * _E)
    comb2 = _build_comb(csid, data_r, w.astype(jnp.float32))
    inp_chunks = jnp.concatenate(
        [inputs[:, :(_NCH - 1) * _CH].reshape(_B, _NCH - 1, _CH),
         jnp.pad(inputs[:, (_NCH - 1) * _CH:_SCROWS],
                 ((0, 0), (0, _CH - _TAILW))).reshape(_B, 1, _CH)], axis=1,
    ).reshape(_B * _NCH, 1, _CH)
    out_partial = _sc_gather()(comb2, inp_chunks)
    idx6 = jnp.pad(inputs[:, _SCROWS:], ((0, 0), (0, 8 - _FIX))
                   ).reshape(_B, 1, 8)
    return _fixup(idx6, comb2, out_partial)


# speedup vs baseline: 9.0410x; 9.0410x over previous
import functools

import jax
import jax.numpy as jnp
from jax import lax
from jax.experimental import pallas as pl
from jax.experimental.pallas import tpu as pltpu
from jax.experimental.pallas import tpu_sc as plsc

_E = 768
_P = 166
_PP = 168
_B = 16
_S = 1126
_W = 5
_CH = 64
_NCH = 18
_NW = 32
_PERW = (_B * _NCH) // _NW


def _comb_kernel(snip_ref, w_ref, out_ref):
    snip = snip_ref[0]
    wt = w_ref[...]
    shifted = (
        jnp.concatenate([snip[:1], snip[:1], snip[:-2]], axis=0),
        jnp.concatenate([snip[:1], snip[:-1]], axis=0),
        snip,
        jnp.concatenate([snip[1:], snip[-1:]], axis=0),
        jnp.concatenate([snip[2:], snip[-1:], snip[-1:]], axis=0),
    )
    acc = shifted[0] * wt[0:1, :]
    for k in range(1, _W):
        acc = acc + shifted[k] * wt[k:k + 1, :]
    out_ref[...] = jnp.concatenate([acc, acc[-2:]], axis=0)


def _build_comb(snips, w):
    return pl.pallas_call(
        _comb_kernel,
        grid=(_B,),
        in_specs=[
            pl.BlockSpec((1, _P, _E), lambda b: (b, 0, 0)),
            pl.BlockSpec((_W, _E), lambda b: (0, 0)),
        ],
        out_specs=pl.BlockSpec((_PP, _E), lambda b: (b, 0)),
        out_shape=jax.ShapeDtypeStruct((_B * _PP, _E), jnp.float32),
    )(snips, w)


def _sc_gather_body(comb_hbm, inpc_hbm, out_hbm, idx_v, gidx_v,
                    oidx0, oidx1, rows0, rows1, gsem, wsem0, wsem1):
    cid = lax.axis_index("c")
    sid = lax.axis_index("s")
    wid = cid * 16 + sid
    b = wid // 2
    c0 = (wid % 2) * _PERW
    bufs = (rows0, rows1)
    oidxs = (oidx0, oidx1)
    wsems = (wsem0, wsem1)
    write_cps = []
    for i in range(_PERW):
        c = c0 + i
        s0 = jnp.minimum(c * _CH, _S - _CH)
        buf = bufs[i % 2]
        oix = oidxs[i % 2]
        if i >= 2:
            write_cps[i - 2].wait()
        pltpu.sync_copy(inpc_hbm.at[b * _NCH + c], idx_v)
        for j in range(_CH // 16):
            v = idx_v[0, pl.ds(j * 16, 16)]
            gidx_v[pl.ds(j * 16, 16)] = jnp.clip(v, 0, _P - 1) + b * _PP
            sv = s0 + j * 16 + lax.iota(jnp.int32, 16)
            oix[pl.ds(j * 16, 16)] = sv * _B + b
        pltpu.async_copy(comb_hbm.at[gidx_v], buf, gsem).wait()
        cp = pltpu.make_async_copy(buf, out_hbm.at[oix], wsems[i % 2])
        cp.start()
        write_cps.append(cp)
    write_cps[-2].wait()
    write_cps[-1].wait()


_sc_gather_cache = []


def _sc_gather():
    if not _sc_gather_cache:
        _sc_gather_cache.append(functools.partial(
            pl.kernel,
            out_type=jax.ShapeDtypeStruct((_S * _B, _E), jnp.float32),
            mesh=plsc.VectorSubcoreMesh(core_axis_name="c", subcore_axis_name="s"),
            scratch_types=[
                pltpu.VMEM((1, _CH), jnp.int32),
                pltpu.VMEM((_CH,), jnp.int32),
                pltpu.VMEM((_CH,), jnp.int32),
                pltpu.VMEM((_CH,), jnp.int32),
                pltpu.VMEM((_CH, _E), jnp.float32),
                pltpu.VMEM((_CH, _E), jnp.float32),
                pltpu.SemaphoreType.DMA,
                pltpu.SemaphoreType.DMA,
                pltpu.SemaphoreType.DMA,
            ],
        )(_sc_gather_body))
    return _sc_gather_cache[0]


def kernel(inputs, code_snippet_id, data, w):
    inputs = inputs.astype(jnp.int32)
    csid = code_snippet_id.astype(jnp.int32).reshape(_B)
    snips = data[csid, :, 0, :]
    comb2 = _build_comb(snips, w.astype(jnp.float32))
    inp_chunks = jnp.concatenate(
        [inputs[:, :(_NCH - 1) * _CH].reshape(_B, _NCH - 1, _CH),
         inputs[:, _S - _CH:].reshape(_B, 1, _CH)], axis=1,
    ).reshape(_B * _NCH, 1, _CH)
    out2 = _sc_gather()(comb2, inp_chunks)
    return out2.reshape(_S, _B, _E).transpose(1, 0, 2)


# speedup vs baseline: 10.8338x; 1.1983x over previous
import functools

import jax
import jax.numpy as jnp
from jax import lax
from jax.experimental import pallas as pl
from jax.experimental.pallas import tpu as pltpu
from jax.experimental.pallas import tpu_sc as plsc

_E = 768
_P = 166
_PP = 168
_B = 16
_S = 1126
_W = 5
_CH = 64
_NCH = 18
_NW = 32
_PERW = (_B * _NCH) // _NW


def _comb_kernel(snip_ref, w_ref, out_ref):
    snip = snip_ref[...]
    wt = w_ref[...]
    shifted = (
        jnp.concatenate([snip[:1], snip[:1], snip[:-2]], axis=0),
        jnp.concatenate([snip[:1], snip[:-1]], axis=0),
        snip,
        jnp.concatenate([snip[1:], snip[-1:]], axis=0),
        jnp.concatenate([snip[2:], snip[-1:], snip[-1:]], axis=0),
    )
    acc = shifted[0] * wt[0:1]
    for k in range(1, _W):
        acc = acc + shifted[k] * wt[k:k + 1]
    out_ref[...] = jnp.concatenate([acc, acc[-2:]], axis=0)


def _build_comb(snips_t, w3):
    return pl.pallas_call(
        _comb_kernel,
        grid=(2,),
        in_specs=[
            pl.BlockSpec((_P, _B // 2, _E), lambda g: (0, g, 0)),
            pl.BlockSpec((_W, 1, _E), lambda g: (0, 0, 0)),
        ],
        out_specs=pl.BlockSpec((_PP, _B // 2, _E), lambda g: (0, g, 0)),
        out_shape=jax.ShapeDtypeStruct((_PP, _B, _E), jnp.float32),
    )(snips_t, w3)


def _sc_gather_body(comb_hbm, inpc_hbm, out_hbm, idx_v, gidx_v,
                    oidx0, oidx1, rows0, rows1, gsem, wsem0, wsem1):
    cid = lax.axis_index("c")
    sid = lax.axis_index("s")
    wid = cid * 16 + sid
    b = wid // 2
    c0 = (wid % 2) * _PERW
    bufs = (rows0, rows1)
    oidxs = (oidx0, oidx1)
    wsems = (wsem0, wsem1)
    write_cps = []
    for i in range(_PERW):
        c = c0 + i
        s0 = jnp.minimum(c * _CH, _S - _CH)
        buf = bufs[i % 2]
        oix = oidxs[i % 2]
        if i >= 2:
            write_cps[i - 2].wait()
        pltpu.sync_copy(inpc_hbm.at[b * _NCH + c], idx_v)
        for j in range(_CH // 16):
            v = idx_v[0, pl.ds(j * 16, 16)]
            gidx_v[pl.ds(j * 16, 16)] = jnp.clip(v, 0, _P - 1) * _B + b
            sv = s0 + j * 16 + lax.iota(jnp.int32, 16)
            oix[pl.ds(j * 16, 16)] = sv * _B + b
        pltpu.async_copy(comb_hbm.at[gidx_v], buf, gsem).wait()
        cp = pltpu.make_async_copy(buf, out_hbm.at[oix], wsems[i % 2])
        cp.start()
        write_cps.append(cp)
    write_cps[-2].wait()
    write_cps[-1].wait()


_sc_gather_cache = []


def _sc_gather():
    if not _sc_gather_cache:
        _sc_gather_cache.append(functools.partial(
            pl.kernel,
            out_type=jax.ShapeDtypeStruct((_S * _B, _E), jnp.float32),
            mesh=plsc.VectorSubcoreMesh(core_axis_name="c", subcore_axis_name="s"),
            scratch_types=[
                pltpu.VMEM((1, _CH), jnp.int32),
                pltpu.VMEM((_CH,), jnp.int32),
                pltpu.VMEM((_CH,), jnp.int32),
                pltpu.VMEM((_CH,), jnp.int32),
                pltpu.VMEM((_CH, _E), jnp.float32),
                pltpu.VMEM((_CH, _E), jnp.float32),
                pltpu.SemaphoreType.DMA,
                pltpu.SemaphoreType.DMA,
                pltpu.SemaphoreType.DMA,
            ],
        )(_sc_gather_body))
    return _sc_gather_cache[0]


def kernel(inputs, code_snippet_id, data, w):
    inputs = inputs.astype(jnp.int32)
    csid = code_snippet_id.astype(jnp.int32).reshape(_B)
    snips = data[csid, :, 0, :]
    snips_t = snips.transpose(1, 0, 2)
    comb3 = _build_comb(snips_t, w.astype(jnp.float32).reshape(_W, 1, _E))
    comb2 = comb3.reshape(_PP * _B, _E)
    inp_chunks = jnp.concatenate(
        [inputs[:, :(_NCH - 1) * _CH].reshape(_B, _NCH - 1, _CH),
         inputs[:, _S - _CH:].reshape(_B, 1, _CH)], axis=1,
    ).reshape(_B * _NCH, 1, _CH)
    out2 = _sc_gather()(comb2, inp_chunks)
    return out2.reshape(_S, _B, _E).transpose(1, 0, 2)


# speedup vs baseline: 10.8448x; 1.0010x over previous
import functools

import jax
import jax.numpy as jnp
from jax import lax
from jax.experimental import pallas as pl
from jax.experimental.pallas import tpu as pltpu
from jax.experimental.pallas import tpu_sc as plsc

_E = 768
_P = 166
_PP = 168
_B = 16
_S = 1126
_W = 5
_CH = 64
_NCH = 18
_NW = 32
_PERW = (_B * _NCH) // _NW


def _comb_kernel(snip_ref, w_ref, out_ref):
    snip = snip_ref[...]
    wt = w_ref[...]
    shifted = (
        jnp.concatenate([snip[:1], snip[:1], snip[:-2]], axis=0),
        jnp.concatenate([snip[:1], snip[:-1]], axis=0),
        snip,
        jnp.concatenate([snip[1:], snip[-1:]], axis=0),
        jnp.concatenate([snip[2:], snip[-1:], snip[-1:]], axis=0),
    )
    acc = shifted[0] * wt[0:1]
    for k in range(1, _W):
        acc = acc + shifted[k] * wt[k:k + 1]
    out_ref[...] = jnp.concatenate([acc, acc[-2:]], axis=0)


def _build_comb(snips_t, w3):
    return pl.pallas_call(
        _comb_kernel,
        grid=(2,),
        in_specs=[
            pl.BlockSpec((_P, _B // 2, _E), lambda g: (0, g, 0)),
            pl.BlockSpec((_W, 1, _E), lambda g: (0, 0, 0)),
        ],
        out_specs=pl.BlockSpec((_PP, _B // 2, _E), lambda g: (0, g, 0)),
        out_shape=jax.ShapeDtypeStruct((_PP, _B, _E), jnp.float32),
    )(snips_t, w3)


def _sc_gather_body(comb_hbm, inpc_hbm, out_hbm, idx_v, gidx_v,
                    oidx0, oidx1, rows0, rows1, gsem, wsem0, wsem1):
    cid = lax.axis_index("c")
    sid = lax.axis_index("s")
    wid = cid * 16 + sid
    b = wid // 2
    c0 = (wid % 2) * _PERW
    bufs = (rows0, rows1)
    oidxs = (oidx0, oidx1)
    wsems = (wsem0, wsem1)
    write_cps = []
    pltpu.sync_copy(inpc_hbm.at[pl.ds(b * _NCH + c0, _PERW)], idx_v)
    for i in range(_PERW):
        c = c0 + i
        s0 = jnp.minimum(c * _CH, _S - _CH)
        buf = bufs[i % 2]
        oix = oidxs[i % 2]
        if i >= 2:
            write_cps[i - 2].wait()
        for j in range(_CH // 16):
            v = idx_v[i, 0, pl.ds(j * 16, 16)]
            gidx_v[pl.ds(j * 16, 16)] = jnp.clip(v, 0, _P - 1) * _B + b
            sv = s0 + j * 16 + lax.iota(jnp.int32, 16)
            oix[pl.ds(j * 16, 16)] = sv * _B + b
        pltpu.async_copy(comb_hbm.at[gidx_v], buf, gsem).wait()
        cp = pltpu.make_async_copy(buf, out_hbm.at[oix], wsems[i % 2])
        cp.start()
        write_cps.append(cp)
    write_cps[-2].wait()
    write_cps[-1].wait()


_sc_gather_cache = []


def _sc_gather():
    if not _sc_gather_cache:
        _sc_gather_cache.append(functools.partial(
            pl.kernel,
            out_type=jax.ShapeDtypeStruct((_S * _B, _E), jnp.float32),
            mesh=plsc.VectorSubcoreMesh(core_axis_name="c", subcore_axis_name="s"),
            scratch_types=[
                pltpu.VMEM((_PERW, 1, _CH), jnp.int32),
                pltpu.VMEM((_CH,), jnp.int32),
                pltpu.VMEM((_CH,), jnp.int32),
                pltpu.VMEM((_CH,), jnp.int32),
                pltpu.VMEM((_CH, _E), jnp.float32),
                pltpu.VMEM((_CH, _E), jnp.float32),
                pltpu.SemaphoreType.DMA,
                pltpu.SemaphoreType.DMA,
                pltpu.SemaphoreType.DMA,
            ],
        )(_sc_gather_body))
    return _sc_gather_cache[0]


def kernel(inputs, code_snippet_id, data, w):
    inputs = inputs.astype(jnp.int32)
    csid = code_snippet_id.astype(jnp.int32).reshape(_B)
    snips = data[csid, :, 0, :]
    snips_t = snips.transpose(1, 0, 2)
    comb3 = _build_comb(snips_t, w.astype(jnp.float32).reshape(_W, 1, _E))
    comb2 = comb3.reshape(_PP * _B, _E)
    inp_chunks = jnp.concatenate(
        [inputs[:, :(_NCH - 1) * _CH].reshape(_B, _NCH - 1, _CH),
         inputs[:, _S - _CH:].reshape(_B, 1, _CH)], axis=1,
    ).reshape(_B * _NCH, 1, _CH)
    out2 = _sc_gather()(comb2, inp_chunks)
    return out2.reshape(_S, _B, _E).transpose(1, 0, 2)
